# Initial kernel scaffold; baseline (speedup 1.0000x reference)
import jax
import jax.numpy as jnp
from jax.experimental import pallas as pl


def kernel(feature, weight, bias, labels):
    raise NotImplementedError("write your pallas kernel here")


# speedup vs baseline: 1.0409x; 1.0409x over previous
import functools

import jax
import jax.numpy as jnp
from jax import lax
from jax.experimental import pallas as pl
from jax.experimental.pallas import tpu as pltpu

_NEG_PAD = -1e30


def _round_up(x, m):
    return ((x + m - 1) // m) * m


def _head_kernel(feat_ref, w_ref, b_ref, labels_ref, out_ref,
                 *, n_rows, tile_n, num_class, lanes):
    pid = pl.program_id(0)
    C = num_class
    L = lanes

    feat = feat_ref[...]
    w = w_ref[...]
    b = b_ref[...]
    labels = labels_ref[...]

    logits = lax.dot_general(
        feat, w, dimension_numbers=(((1,), (0,)), ((), ())),
        preferred_element_type=jnp.float32) + b
    TN = logits.shape[0]

    row = lax.broadcasted_iota(jnp.int32, (TN, 1), 0)
    real = (pid * tile_n + row) < n_rows
    valid = (labels >= 0) & real

    col = lax.broadcasted_iota(jnp.int32, (TN, L), 1)
    adj = jnp.where(labels < 0, labels + C, labels)

    m = jnp.max(logits, axis=1, keepdims=True)
    lse = m + jnp.log(jnp.sum(jnp.exp(logits - m), axis=1, keepdims=True))

    oh = (col == adj) & real
    ohf = oh.astype(jnp.float32)
    logit_at = jnp.sum(jnp.where(oh, logits, 0.0), axis=1, keepdims=True)
    per_row_loss = jnp.where(valid, lse - logit_at, 0.0)

    preds = jnp.min(jnp.where(logits == m, col, L), axis=1, keepdims=True)
    match = preds == labels

    loss_sum = jnp.sum(per_row_loss)
    n_valid = jnp.sum(jnp.where(valid, 1.0, 0.0))
    acc_sum = jnp.sum(jnp.where(valid & match, 1.0, 0.0))

    total_row = jnp.sum(ohf, axis=0, keepdims=True)
    correct_row = jnp.sum(jnp.where(match, ohf, 0.0), axis=0, keepdims=True)

    lane = lax.broadcasted_iota(jnp.int32, (1, L), 1)
    scal = (jnp.where(lane == 0, loss_sum, 0.0)
            + jnp.where(lane == 1, n_valid, 0.0)
            + jnp.where(lane == 2, acc_sum, 0.0))
    sub = lax.broadcasted_iota(jnp.int32, (3, L), 0)
    block = jnp.where(sub == 0, correct_row,
                      jnp.where(sub == 1, total_row, scal))
    out_ref[...] = block.reshape(1, 3, L)


def kernel(feature, weight, bias, labels):
    N, D = feature.shape
    C = weight.shape[0]
    L = max(128, _round_up(C, 128))
    TN = min(1024, _round_up(N, 8))
    num_tiles = pl.cdiv(N, TN)

    w_pad = jnp.pad(weight.T.astype(feature.dtype), ((0, 0), (0, L - C)))
    b_pad = jnp.pad(bias.astype(jnp.float32).reshape(1, C),
                    ((0, 0), (0, L - C)), constant_values=_NEG_PAD)
    labels2d = labels.astype(jnp.int32).reshape(N, 1)

    kernel_fn = functools.partial(_head_kernel, n_rows=N, tile_n=TN,
                                  num_class=C, lanes=L)

    part = pl.pallas_call(
        kernel_fn,
        grid=(num_tiles,),
        in_specs=[
            pl.BlockSpec((TN, D), lambda i: (i, 0)),
            pl.BlockSpec((D, L), lambda i: (0, 0)),
            pl.BlockSpec((1, L), lambda i: (0, 0)),
            pl.BlockSpec((TN, 1), lambda i: (i, 0)),
        ],
        out_specs=pl.BlockSpec((1, 3, L), lambda i: (i, 0, 0)),
        out_shape=jax.ShapeDtypeStruct((num_tiles, 3, L), jnp.float32),
        compiler_params=pltpu.CompilerParams(
            dimension_semantics=("parallel",),
            vmem_limit_bytes=48 * 1024 * 1024,
        ),
    )(feature, w_pad, b_pad, labels2d)

    part = jnp.sum(part, axis=0)
    correct = part[0, :C]
    total = part[1, :C]
    loss_sum = part[2, 0]
    n_valid = part[2, 1]
    acc_sum = part[2, 2]

    loss = loss_sum / n_valid
    acc = acc_sum / (n_valid + 1e-10)
    cat = jnp.stack([correct, total], axis=0)
    return loss, acc, cat


# speedup vs baseline: 1.1340x; 1.0894x over previous
import functools

import jax
import jax.numpy as jnp
from jax import lax
from jax.experimental import pallas as pl
from jax.experimental.pallas import tpu as pltpu

_NEG_PAD = -1e30


def _round_up(x, m):
    return ((x + m - 1) // m) * m


def _head_kernel(feat_ref, w_ref, b_ref, labels_ref, out_ref,
                 *, n_rows, tile_n, num_class, lanes):
    C = num_class
    L = lanes
    aligned = (n_rows % tile_n == 0)

    feat = feat_ref[...]
    w = w_ref[...]
    b = b_ref[...]
    labels = labels_ref[...]

    logits = lax.dot_general(
        feat, w, dimension_numbers=(((1,), (0,)), ((), ())),
        preferred_element_type=jnp.float32) + b
    TN = logits.shape[0]

    if aligned:
        valid = labels >= 0
    else:
        row = lax.broadcasted_iota(jnp.int32, (TN, 1), 0)
        real = (pl.program_id(0) * tile_n + row) < n_rows
        valid = (labels >= 0) & real

    col = lax.broadcasted_iota(jnp.int32, (TN, L), 1)
    adj = jnp.where(labels < 0, labels + C, labels)

    m = jnp.max(logits, axis=1, keepdims=True)
    lse = m + jnp.log(jnp.sum(jnp.exp(logits - m), axis=1, keepdims=True))

    oh = col == adj
    if not aligned:
        oh = oh & real
    ohf = jnp.where(oh, 1.0, 0.0)
    logit_at = jnp.sum(ohf * logits, axis=1, keepdims=True)
    per_row_loss = jnp.where(valid, lse - logit_at, 0.0)

    preds = jnp.min(jnp.where(logits == m, col, L), axis=1, keepdims=True)
    match = preds == labels
    matchf = jnp.where(match, 1.0, 0.0)

    lane8 = lax.broadcasted_iota(jnp.int32, (TN, 8), 1)
    mm = jnp.where(lane8 == 0, 1.0, 0.0) + jnp.where(lane8 == 1, matchf, 0.0)
    cnt = lax.dot_general(
        mm, ohf, dimension_numbers=(((0,), (0,)), ((), ())),
        preferred_element_type=jnp.float32)

    loss_sum = jnp.sum(per_row_loss)
    n_valid = jnp.sum(jnp.where(valid, 1.0, 0.0))
    if aligned:
        acc_sum = jnp.sum(matchf)
    else:
        acc_sum = jnp.sum(jnp.where(valid & match, 1.0, 0.0))

    lane = lax.broadcasted_iota(jnp.int32, (1, L), 1)
    scal = (jnp.where(lane == 0, loss_sum, 0.0)
            + jnp.where(lane == 1, n_valid, 0.0)
            + jnp.where(lane == 2, acc_sum, 0.0))
    sub = lax.broadcasted_iota(jnp.int32, (8, L), 0)
    block = cnt + jnp.where(sub == 2, scal, 0.0)
    out_ref[...] = block.reshape(1, 8, L)


def kernel(feature, weight, bias, labels):
    N, D = feature.shape
    C = weight.shape[0]
    L = max(128, _round_up(C, 128))
    TN = min(1024, _round_up(N, 8))
    num_tiles = pl.cdiv(N, TN)

    w_pad = jnp.pad(weight.T.astype(feature.dtype), ((0, 0), (0, L - C)))
    b_pad = jnp.pad(bias.astype(jnp.float32).reshape(1, C),
                    ((0, 0), (0, L - C)), constant_values=_NEG_PAD)
    labels2d = labels.astype(jnp.int32).reshape(N, 1)

    kernel_fn = functools.partial(_head_kernel, n_rows=N, tile_n=TN,
                                  num_class=C, lanes=L)

    part = pl.pallas_call(
        kernel_fn,
        grid=(num_tiles,),
        in_specs=[
            pl.BlockSpec((TN, D), lambda i: (i, 0)),
            pl.BlockSpec((D, L), lambda i: (0, 0)),
            pl.BlockSpec((1, L), lambda i: (0, 0)),
            pl.BlockSpec((TN, 1), lambda i: (i, 0)),
        ],
        out_specs=pl.BlockSpec((1, 8, L), lambda i: (i, 0, 0)),
        out_shape=jax.ShapeDtypeStruct((num_tiles, 8, L), jnp.float32),
        compiler_params=pltpu.CompilerParams(
            dimension_semantics=("parallel",),
            vmem_limit_bytes=48 * 1024 * 1024,
        ),
    )(feature, w_pad, b_pad, labels2d)

    part = jnp.sum(part, axis=0)
    total = part[0, :C]
    correct = part[1, :C]
    loss_sum = part[2, 0]
    n_valid = part[2, 1]
    acc_sum = part[2, 2]

    loss = loss_sum / n_valid
    acc = acc_sum / (n_valid + 1e-10)
    cat = jnp.stack([correct, total], axis=0)
    return loss, acc, cat


# speedup vs baseline: 1.2099x; 1.0669x over previous
import functools

import jax
import jax.numpy as jnp
from jax import lax
from jax.experimental import pallas as pl
from jax.experimental.pallas import tpu as pltpu

_NEG_PAD = -1e30
_LOG2E = 1.4426950408889634


def _round_up(x, m):
    return ((x + m - 1) // m) * m


def _head_kernel(feat_ref, w_ref, b_ref, labels_ref, out_ref,
                 *, n_rows, tile_n, num_class, lanes):
    C = num_class
    L = lanes
    aligned = (n_rows % tile_n == 0)

    feat = feat_ref[...]
    w = w_ref[...]
    b = b_ref[...]
    labels = labels_ref[...]

    logits = lax.dot_general(
        feat, w, dimension_numbers=(((1,), (0,)), ((), ())),
        preferred_element_type=jnp.float32) + b
    TN = logits.shape[0]

    if aligned:
        valid = labels >= 0
    else:
        row = lax.broadcasted_iota(jnp.int32, (TN, 1), 0)
        real = (pl.program_id(0) * tile_n + row) < n_rows
        valid = (labels >= 0) & real

    colf = lax.broadcasted_iota(jnp.int32, (TN, L), 1).astype(jnp.float32)
    adj = jnp.where(labels < 0, labels + C, labels)
    adjf = adj.astype(jnp.float32)
    labelsf = labels.astype(jnp.float32)

    m = jnp.max(logits, axis=1, keepdims=True)
    ms = m * _LOG2E
    se = jnp.sum(jnp.exp2(logits * _LOG2E - ms), axis=1, keepdims=True)
    lse = m + jnp.log(se)

    oh = colf == adjf
    if not aligned:
        oh = oh & real
    logit_at = jnp.sum(jnp.where(oh, logits, 0.0), axis=1, keepdims=True)
    per_row_loss = jnp.where(valid, lse - logit_at, 0.0)

    predsf = jnp.min(jnp.where(logits == m, colf, float(L)),
                     axis=1, keepdims=True)
    match = predsf == labelsf
    matchf = jnp.where(match, 1.0, 0.0)
    validf = jnp.where(valid, 1.0, 0.0)

    lane8 = lax.broadcasted_iota(jnp.int32, (TN, 8), 1)
    mm = (jnp.where(lane8 == 0, 1.0, 0.0)
          + jnp.where(lane8 == 1, matchf, 0.0)
          + jnp.where(lane8 == 2, validf, 0.0))
    cnt = lax.dot_general(
        mm, jnp.where(oh, 1.0, 0.0),
        dimension_numbers=(((0,), (0,)), ((), ())),
        preferred_element_type=jnp.float32)

    loss_sum = jnp.sum(per_row_loss)
    lane = lax.broadcasted_iota(jnp.int32, (1, L), 1)
    sub = lax.broadcasted_iota(jnp.int32, (8, L), 0)
    block = cnt + jnp.where((sub == 3) & (lane == 0), loss_sum, 0.0)
    out_ref[...] = block.reshape(1, 8, L)


def kernel(feature, weight, bias, labels):
    N, D = feature.shape
    C = weight.shape[0]
    L = max(128, _round_up(C, 128))
    TN = min(1024, _round_up(N, 8))
    num_tiles = pl.cdiv(N, TN)

    w_pad = jnp.pad(weight.T.astype(feature.dtype), ((0, 0), (0, L - C)))
    b_pad = jnp.pad(bias.astype(jnp.float32).reshape(1, C),
                    ((0, 0), (0, L - C)), constant_values=_NEG_PAD)
    labels2d = labels.astype(jnp.int32).reshape(N, 1)

    kernel_fn = functools.partial(_head_kernel, n_rows=N, tile_n=TN,
                                  num_class=C, lanes=L)

    part = pl.pallas_call(
        kernel_fn,
        grid=(num_tiles,),
        in_specs=[
            pl.BlockSpec((TN, D), lambda i: (i, 0)),
            pl.BlockSpec((D, L), lambda i: (0, 0)),
            pl.BlockSpec((1, L), lambda i: (0, 0)),
            pl.BlockSpec((TN, 1), lambda i: (i, 0)),
        ],
        out_specs=pl.BlockSpec((1, 8, L), lambda i: (i, 0, 0)),
        out_shape=jax.ShapeDtypeStruct((num_tiles, 8, L), jnp.float32),
        compiler_params=pltpu.CompilerParams(
            dimension_semantics=("parallel",),
            vmem_limit_bytes=48 * 1024 * 1024,
        ),
    )(feature, w_pad, b_pad, labels2d)

    part = jnp.sum(part, axis=0)
    total = part[0, :C]
    correct = part[1, :C]
    n_valid = jnp.sum(part[2])
    acc_sum = jnp.sum(correct)
    loss_sum = part[3, 0]

    loss = loss_sum / n_valid
    acc = acc_sum / (n_valid + 1e-10)
    cat = jnp.stack([correct, total], axis=0)
    return loss, acc, cat
